# baseline (speedup 1.0000x reference)
import functools

import jax
import jax.numpy as jnp
from jax import lax
from jax.experimental import pallas as pl
from jax.experimental.pallas import tpu as pltpu
from jax.experimental.pallas import tpu_sc as plsc

_D = 64
_LANES = 16


def _sc_weights_body(idx_hbm, freq_hbm, out_hbm, idx_v, f_v, w_v, sem, *,
                     l_pad):
    cid = lax.axis_index("c")
    sid = lax.axis_index("s")

    @pl.when(jnp.logical_and(cid == 0, sid == 0))
    def _():
        pltpu.sync_copy(idx_hbm, idx_v)
        pltpu.async_copy(freq_hbm.at[idx_v], f_v, sem).wait()
        for k in range(l_pad // _LANES):
            sl = pl.ds(k * _LANES, _LANES)
            w_v[sl] = 1.0 / f_v[sl]
        pltpu.sync_copy(w_v, out_hbm)


def _make_sc_weights(l_pad):
    return functools.partial(
        pl.kernel,
        out_type=jax.ShapeDtypeStruct((l_pad,), jnp.float32),
        mesh=plsc.VectorSubcoreMesh(core_axis_name="c", subcore_axis_name="s"),
        scratch_types=[
            pltpu.VMEM((l_pad,), jnp.int32),
            pltpu.VMEM((l_pad,), jnp.float32),
            pltpu.VMEM((l_pad,), jnp.float32),
            pltpu.SemaphoreType.DMA,
        ],
        compiler_params=pltpu.CompilerParams(use_tc_tiling_on_sc=True),
    )(functools.partial(_sc_weights_body, l_pad=l_pad))


def _tc_body(idx_ref, w_ref, wt_ref, b_ref, emb_ref, out_ref,
             rows_v, sems, *, l_pad, scale):
    nsem = sems.shape[0]
    cps = []
    for j in range(l_pad):
        r = idx_ref[j]
        cps.append(pltpu.make_async_copy(
            emb_ref.at[pl.ds(r, 1), :], rows_v.at[pl.ds(j, 1), :],
            sems.at[j % nsem]))
    for cp in cps:
        cp.start()
    for cp in cps:
        cp.wait()

    bow = lax.dot_general(
        w_ref[...], rows_v[...], (((1,), (0,)), ((), ())),
        preferred_element_type=jnp.float32) * scale
    logits = lax.dot_general(
        bow, wt_ref[...], (((1,), (1,)), ((), ())),
        preferred_element_type=jnp.float32) + b_ref[...]
    m = jnp.max(logits, axis=-1, keepdims=True)
    s = logits - m
    lse = jnp.log(jnp.sum(jnp.exp(s), axis=-1, keepdims=True))
    out_ref[...] = s - lse


def kernel(input, emb_tensor, freq, W, b):
    L = input.shape[1]
    l_pad = ((L + _LANES - 1) // _LANES) * _LANES
    idx = jnp.concatenate(
        [input[0], jnp.zeros((l_pad - L,), jnp.int32)])
    w = _make_sc_weights(l_pad)(idx, freq)

    scale = 1.0 / (float(L) * 100000.0)
    out = pl.pallas_call(
        functools.partial(_tc_body, l_pad=l_pad, scale=scale),
        out_shape=jax.ShapeDtypeStruct((1, 2), jnp.float32),
        in_specs=[
            pl.BlockSpec(memory_space=pltpu.SMEM),
            pl.BlockSpec(memory_space=pltpu.VMEM),
            pl.BlockSpec(memory_space=pltpu.VMEM),
            pl.BlockSpec(memory_space=pltpu.VMEM),
            pl.BlockSpec(memory_space=pltpu.MemorySpace.HBM),
        ],
        out_specs=pl.BlockSpec(memory_space=pltpu.VMEM),
        scratch_shapes=[
            pltpu.VMEM((l_pad, _D), jnp.float32),
            pltpu.SemaphoreType.DMA((16,)),
        ],
    )(idx, w.reshape(1, l_pad), W, b.reshape(1, 2), emb_tensor)
    return out
